# baseline (speedup 1.0000x reference)
import functools

import jax
import jax.numpy as jnp
from jax import lax
from jax.experimental import pallas as pl
from jax.experimental.pallas import tpu as pltpu
from jax.experimental.pallas import tpu_sc as plsc

D_MODEL = 1024
N_KINDS = 2
LANES = 16
D_VECS = D_MODEL // LANES


def _make_sc_kernel(num_tokens):
    info = plsc.get_sparse_core_info()
    nc, ns = info.num_cores, info.num_subcores
    nw = nc * ns
    tok_per_w = num_tokens // nw
    chunk = 64
    n_chunks = tok_per_w // chunk

    mesh = plsc.VectorSubcoreMesh(core_axis_name="c", subcore_axis_name="s")

    @functools.partial(
        pl.kernel,
        mesh=mesh,
        out_type=jax.ShapeDtypeStruct((num_tokens, D_MODEL), jnp.float32),
        scratch_types=[
            pltpu.VMEM((chunk,), jnp.int32),
            pltpu.VMEM((chunk,), jnp.int32),
            pltpu.VMEM((chunk, D_MODEL), jnp.float32),
            pltpu.VMEM((N_KINDS, D_MODEL), jnp.float32),
            pltpu.VMEM((1, D_MODEL), jnp.float32),
            pltpu.SemaphoreType.DMA,
        ],
    )
    def sc_kernel(t_hbm, kind_hbm, time_hbm, view_hbm, kind_emb_hbm, out_hbm,
                  t_idx, k_idx, buf, comb, viewv, sem):
        wid = lax.axis_index("s") * nc + lax.axis_index("c")
        w_base = wid * tok_per_w

        pltpu.sync_copy(kind_emb_hbm, comb)
        pltpu.sync_copy(view_hbm, viewv)
        for d in range(D_VECS):
            vv = viewv[0, pl.ds(d * LANES, LANES)]
            for k in range(N_KINDS):
                plsc.addupdate(comb.at[k, pl.ds(d * LANES, LANES)], vv)

        def chunk_body(ci, carry):
            base = w_base + ci * chunk
            pltpu.sync_copy(t_hbm.at[pl.ds(base, chunk)], t_idx)
            pltpu.sync_copy(kind_hbm.at[pl.ds(base, chunk)], k_idx)
            pltpu.async_copy(time_hbm.at[t_idx], buf, sem).wait()

            def group_body(g, c2):
                kvec = k_idx[pl.ds(g * LANES, LANES)]
                for l in range(LANES):
                    kj = kvec[l]
                    j = g * LANES + l
                    for d in range(D_VECS):
                        sl = pl.ds(d * LANES, LANES)
                        plsc.addupdate(buf.at[j, sl], comb[kj, sl])
                return c2

            lax.fori_loop(0, chunk // LANES, group_body, 0, unroll=False)
            pltpu.sync_copy(buf, out_hbm.at[pl.ds(base, chunk)])
            return carry

        lax.fori_loop(0, n_chunks, chunk_body, 0, unroll=False)

    return sc_kernel


def kernel(t, kind_id, view_id, time_emb, view_emb, kind_emb):
    b, s = t.shape
    n = b * s
    t_flat = t.reshape(n).astype(jnp.int32)
    kind_flat = kind_id.reshape(n).astype(jnp.int32)
    del view_id
    out = _make_sc_kernel(n)(t_flat, kind_flat, time_emb, view_emb, kind_emb)
    return out.reshape(b, s, D_MODEL)


# speedup vs baseline: 1.9228x; 1.9228x over previous
import functools

import jax
import jax.numpy as jnp
from jax import lax
from jax.experimental import pallas as pl
from jax.experimental.pallas import tpu as pltpu
from jax.experimental.pallas import tpu_sc as plsc

D_MODEL = 1024
N_KINDS = 2
LANES = 16
D_VECS = D_MODEL // LANES
CHUNK = 16
NBUF = 4


def _make_sc_kernel(num_tokens):
    info = plsc.get_sparse_core_info()
    nc, ns = info.num_cores, info.num_subcores
    nw = nc * ns
    tok_per_w = num_tokens // nw
    n_chunks = tok_per_w // CHUNK
    n_groups = n_chunks // NBUF

    mesh = plsc.VectorSubcoreMesh(core_axis_name="c", subcore_axis_name="s")

    @functools.partial(
        pl.kernel,
        mesh=mesh,
        out_type=jax.ShapeDtypeStruct((num_tokens, D_MODEL), jnp.float32),
        scratch_types=[
            pltpu.VMEM((tok_per_w,), jnp.int32),
            pltpu.VMEM((tok_per_w,), jnp.int32),
            pltpu.VMEM((CHUNK, D_MODEL), jnp.float32),
            pltpu.VMEM((CHUNK, D_MODEL), jnp.float32),
            pltpu.VMEM((CHUNK, D_MODEL), jnp.float32),
            pltpu.VMEM((CHUNK, D_MODEL), jnp.float32),
            pltpu.VMEM((N_KINDS, D_MODEL), jnp.float32),
            pltpu.VMEM((1, D_MODEL), jnp.float32),
            pltpu.SemaphoreType.DMA,
            pltpu.SemaphoreType.DMA,
            pltpu.SemaphoreType.DMA,
            pltpu.SemaphoreType.DMA,
            pltpu.SemaphoreType.DMA,
            pltpu.SemaphoreType.DMA,
            pltpu.SemaphoreType.DMA,
            pltpu.SemaphoreType.DMA,
        ],
    )
    def sc_kernel(t_hbm, kind_hbm, time_hbm, view_hbm, kind_emb_hbm, out_hbm,
                  t_all, k_all, b0, b1, b2, b3, comb, viewv,
                  g0, g1, g2, g3, w0, w1, w2, w3):
        bufs = (b0, b1, b2, b3)
        gsem = (g0, g1, g2, g3)
        wsem = (w0, w1, w2, w3)
        wid = lax.axis_index("s") * nc + lax.axis_index("c")
        w_base = wid * tok_per_w

        pltpu.sync_copy(t_hbm.at[pl.ds(w_base, tok_per_w)], t_all)
        pltpu.sync_copy(kind_hbm.at[pl.ds(w_base, tok_per_w)], k_all)

        pltpu.sync_copy(kind_emb_hbm, comb)
        pltpu.sync_copy(view_hbm, viewv)
        for d in range(D_VECS):
            vv = viewv[0, pl.ds(d * LANES, LANES)]
            for k in range(N_KINDS):
                plsc.addupdate(comb.at[k, pl.ds(d * LANES, LANES)], vv)

        def gather(c, b):
            tv = t_all[pl.ds(c * CHUNK, CHUNK)]
            return pltpu.async_copy(time_hbm.at[tv], bufs[b], gsem[b])

        gather(0, 0)
        gather(1, 1)

        def group_body(g, carry):
            for u in range(NBUF):
                c = g * NBUF + u
                bg = (u + 2) % NBUF

                @pl.when(c >= 2)
                def _drain():
                    pltpu.make_async_copy(bufs[bg], out_hbm.at[pl.ds(0, CHUNK)],
                                          wsem[bg]).wait()

                @pl.when(c + 2 < n_chunks)
                def _prefetch():
                    gather(c + 2, bg)

                pltpu.make_async_copy(time_hbm.at[t_all[pl.ds(0, CHUNK)]],
                                      bufs[u], gsem[u]).wait()
                kvec = k_all[pl.ds(c * CHUNK, CHUNK)]
                kjs = [kvec[l] for l in range(LANES)]

                def d_body(d, c2, _kjs=kjs, _u=u):
                    sl = pl.ds(d * LANES, LANES)
                    for l in range(LANES):
                        plsc.addupdate(bufs[_u].at[l, sl], comb[_kjs[l], sl])
                    return c2

                lax.fori_loop(0, D_VECS, d_body, 0, unroll=2)
                pltpu.async_copy(bufs[u], out_hbm.at[pl.ds(w_base + c * CHUNK, CHUNK)],
                                 wsem[u])
            return carry

        lax.fori_loop(0, n_groups, group_body, 0, unroll=False)

        pltpu.make_async_copy(b2, out_hbm.at[pl.ds(0, CHUNK)], w2).wait()
        pltpu.make_async_copy(b3, out_hbm.at[pl.ds(0, CHUNK)], w3).wait()

    return sc_kernel


def kernel(t, kind_id, view_id, time_emb, view_emb, kind_emb):
    b, s = t.shape
    n = b * s
    t_flat = t.reshape(n).astype(jnp.int32)
    kind_flat = kind_id.reshape(n).astype(jnp.int32)
    del view_id
    out = _make_sc_kernel(n)(t_flat, kind_flat, time_emb, view_emb, kind_emb)
    return out.reshape(b, s, D_MODEL)


# speedup vs baseline: 4.1041x; 2.1345x over previous
import functools

import jax
import jax.numpy as jnp
from jax import lax
from jax.experimental import pallas as pl
from jax.experimental.pallas import tpu as pltpu
from jax.experimental.pallas import tpu_sc as plsc

D_MODEL = 1024
N_KINDS = 2
LANES = 16
D_VECS = D_MODEL // LANES
CHUNK = 16
NBUF = 4


def _make_sc_kernel(num_tokens):
    info = plsc.get_sparse_core_info()
    nc, ns = info.num_cores, info.num_subcores
    nw = nc * ns
    tok_per_w = num_tokens // nw
    n_chunks = tok_per_w // CHUNK
    n_groups = n_chunks // NBUF

    mesh = plsc.VectorSubcoreMesh(core_axis_name="c", subcore_axis_name="s")

    @functools.partial(
        pl.kernel,
        mesh=mesh,
        out_type=jax.ShapeDtypeStruct((num_tokens, D_MODEL), jnp.float32),
        scratch_types=[
            pltpu.VMEM((tok_per_w,), jnp.int32),
            pltpu.VMEM((tok_per_w,), jnp.int32),
            pltpu.VMEM((CHUNK, D_MODEL), jnp.float32),
            pltpu.VMEM((CHUNK, D_MODEL), jnp.float32),
            pltpu.VMEM((CHUNK, D_MODEL), jnp.float32),
            pltpu.VMEM((CHUNK, D_MODEL), jnp.float32),
            pltpu.VMEM((N_KINDS, D_MODEL), jnp.float32),
            pltpu.VMEM((1, D_MODEL), jnp.float32),
            pltpu.SemaphoreType.DMA,
            pltpu.SemaphoreType.DMA,
            pltpu.SemaphoreType.DMA,
            pltpu.SemaphoreType.DMA,
            pltpu.SemaphoreType.DMA,
            pltpu.SemaphoreType.DMA,
            pltpu.SemaphoreType.DMA,
            pltpu.SemaphoreType.DMA,
        ],
    )
    def sc_kernel(t_hbm, kind_hbm, time_hbm, view_hbm, kind_emb_hbm, out_hbm,
                  t_all, k_all, b0, b1, b2, b3, comb, viewv,
                  g0, g1, g2, g3, w0, w1, w2, w3):
        bufs = (b0, b1, b2, b3)
        gsem = (g0, g1, g2, g3)
        wsem = (w0, w1, w2, w3)
        wid = lax.axis_index("s") * nc + lax.axis_index("c")
        w_base = wid * tok_per_w

        pltpu.sync_copy(t_hbm.at[pl.ds(w_base, tok_per_w)], t_all)
        pltpu.sync_copy(kind_hbm.at[pl.ds(w_base, tok_per_w)], k_all)

        pltpu.sync_copy(kind_emb_hbm, comb)
        pltpu.sync_copy(view_hbm, viewv)
        for d in range(D_VECS):
            vv = viewv[0, pl.ds(d * LANES, LANES)]
            for k in range(N_KINDS):
                plsc.addupdate(comb.at[k, pl.ds(d * LANES, LANES)], vv)

        def gather(c, b):
            tv = t_all[pl.ds(c * CHUNK, CHUNK)]
            return pltpu.async_copy(time_hbm.at[tv], bufs[b], gsem[b])

        gather(0, 0)
        gather(1, 1)

        def group_body(g, carry):
            for u in range(NBUF):
                c = g * NBUF + u
                bg = (u + 2) % NBUF

                @pl.when(c >= 2)
                def _drain():
                    pltpu.make_async_copy(bufs[bg], out_hbm.at[pl.ds(0, CHUNK)],
                                          wsem[bg]).wait()

                @pl.when(c + 2 < n_chunks)
                def _prefetch():
                    gather(c + 2, bg)

                pltpu.make_async_copy(time_hbm.at[t_all[pl.ds(0, CHUNK)]],
                                      bufs[u], gsem[u]).wait()
                kvec = k_all[pl.ds(c * CHUNK, CHUNK)]
                kjs = [kvec[l] for l in range(LANES)]

                def d_body(d, c2, _kjs=kjs, _u=u):
                    sl = pl.ds(d * LANES, LANES)
                    vals = [comb[_kjs[l], sl] for l in range(LANES)]
                    for l in range(LANES):
                        plsc.addupdate(bufs[_u].at[l, sl], vals[l])
                    return c2

                lax.fori_loop(0, D_VECS, d_body, 0, unroll=2)
                pltpu.async_copy(bufs[u], out_hbm.at[pl.ds(w_base + c * CHUNK, CHUNK)],
                                 wsem[u])
            return carry

        lax.fori_loop(0, n_groups, group_body, 0, unroll=False)

        pltpu.make_async_copy(b2, out_hbm.at[pl.ds(0, CHUNK)], w2).wait()
        pltpu.make_async_copy(b3, out_hbm.at[pl.ds(0, CHUNK)], w3).wait()

    return sc_kernel


def kernel(t, kind_id, view_id, time_emb, view_emb, kind_emb):
    b, s = t.shape
    n = b * s
    t_flat = t.reshape(n).astype(jnp.int32)
    kind_flat = kind_id.reshape(n).astype(jnp.int32)
    del view_id
    out = _make_sc_kernel(n)(t_flat, kind_flat, time_emb, view_emb, kind_emb)
    return out.reshape(b, s, D_MODEL)
